# baseline (speedup 1.0000x reference)
import functools

import jax
import jax.numpy as jnp
from jax import lax
from jax.experimental import pallas as pl
from jax.experimental.pallas import tpu as pltpu
from jax.experimental.pallas import tpu_sc as plsc

B, S = 4096, 200
VOCAB = 1000000
ENC_DIM = 16
N = B * S
ROW = 6

PACK = 8
PK = PACK * ROW
PH = PACK * 2 * ENC_DIM
PO = PACK * ENC_DIM
NPACKT = VOCAB // PACK
BVT = 1024
GRIDT = -(-NPACKT // BVT)

NC, NS = 2, 16
NW = NC * NS
PER_W = N // NW
GKEYS = S
NGROUP = PER_W // GKEYS
HSPLIT = (104, 96)
OUTW = S * ENC_DIM

TBR = 512
GRIDF = B // TBR


def _keys_body(node_ref, t_ref, out_ref):
    s = node_ref[...] + t_ref[...].astype(jnp.int32)
    out_ref[...] = jnp.where(s >= VOCAB, s - VOCAB, s)


_keys_call = pl.pallas_call(
    _keys_body,
    out_shape=jax.ShapeDtypeStruct((N // 128, 128), jnp.int32),
)


def _mlp_body(x_ref, w1_ref, b1_ref, w2_ref, b2_ref, o_ref):
    x = x_ref[...]
    h = jnp.maximum(
        jnp.dot(w1_ref[...], x, preferred_element_type=jnp.float32) + b1_ref[...],
        0.0,
    )
    o = jnp.dot(w2_ref[...], h, preferred_element_type=jnp.float32) + b2_ref[...]
    o_ref[...] = jnp.transpose(o)


_tab_call = pl.pallas_call(
    _mlp_body,
    grid=(GRIDT,),
    in_specs=[
        pl.BlockSpec((PK, BVT), lambda i: (0, i)),
        pl.BlockSpec((PH, PK), lambda i: (0, 0)),
        pl.BlockSpec((PH, 1), lambda i: (0, 0)),
        pl.BlockSpec((PO, PH), lambda i: (0, 0)),
        pl.BlockSpec((PO, 1), lambda i: (0, 0)),
    ],
    out_specs=pl.BlockSpec((BVT, PO), lambda i: (i, 0)),
    out_shape=jax.ShapeDtypeStruct((NPACKT, PO), jnp.float32),
)


def _tr_body(x_ref, o_ref):
    o_ref[...] = jnp.transpose(x_ref[...])


_final_tr_call = pl.pallas_call(
    _tr_body,
    grid=(GRIDF,),
    in_specs=[pl.BlockSpec((TBR, OUTW), lambda i: (i, 0))],
    out_specs=pl.BlockSpec((OUTW, TBR), lambda i: (0, i)),
    out_shape=jax.ShapeDtypeStruct((OUTW, B), jnp.float32),
)


_sc_mesh = plsc.VectorSubcoreMesh(core_axis_name="c", subcore_axis_name="s")


@functools.partial(
    pl.kernel,
    out_type=jax.ShapeDtypeStruct((B, OUTW), jnp.float32),
    mesh=_sc_mesh,
    scratch_types=[
        pltpu.VMEM((PER_W,), jnp.int32),
        pltpu.VMEM((2, GKEYS), jnp.int32),
        pltpu.VMEM((2, GKEYS, PO), jnp.float32),
        pltpu.VMEM((1, OUTW), jnp.float32),
        pltpu.SemaphoreType.DMA,
        pltpu.SemaphoreType.DMA,
    ],
    compiler_params=pltpu.CompilerParams(
        use_tc_tiling_on_sc=False, needs_layout_passes=False
    ),
)
def _sc_gather(keys_hbm, ftab_hbm, out_hbm, keys_v, q_v, rows_v, stage_v, sem0, sem1):
    wid = lax.axis_index("s") * NC + lax.axis_index("c")
    key_base = wid * PER_W
    row_base = wid * NGROUP
    sems = (sem0, sem1)

    pltpu.sync_copy(keys_hbm.at[pl.ds(key_base, PER_W)], keys_v)
    iota = lax.iota(jnp.int32, 16)
    iota16 = iota * 16

    offs = [16 * t for t in range(12)] + [GKEYS - 16]

    def fire(g, buf):
        for o in offs:
            kv = keys_v[pl.ds(g * GKEYS + o, 16)]
            q_v[buf, pl.ds(o, 16)] = lax.shift_right_logical(kv, 3)
        o = 0
        for sz in HSPLIT:
            pltpu.async_copy(
                ftab_hbm.at[q_v.at[buf, pl.ds(o, sz)]],
                rows_v.at[buf, pl.ds(o, sz)],
                sems[buf],
            )
            o += sz

    def drain(buf):
        o = 0
        for sz in HSPLIT:
            pltpu.make_async_copy(
                ftab_hbm.at[q_v.at[buf, pl.ds(o, sz)]],
                rows_v.at[buf, pl.ds(o, sz)],
                sems[buf],
            ).wait()
            o += sz

    def repack_and_write(g, buf):
        rows = rows_v.at[buf]
        stage = stage_v.at[0]
        for o in offs:
            kv = keys_v[pl.ds(g * GKEYS + o, 16)]
            lane_base = lax.shift_left(jnp.bitwise_and(kv, 7), 4)
            row_ids = iota + o
            sidx = iota16 + (o * ENC_DIM)
            for j in range(ENC_DIM):
                vals = plsc.load_gather(rows, [row_ids, lane_base + j])
                plsc.store_scatter(stage, [sidx + j], vals)
        pltpu.sync_copy(stage_v, out_hbm.at[pl.ds(row_base + g, 1)])

    fire(0, 0)
    fire(1, 1)

    def pipe_body(g2, carry):
        g0 = 2 * g2
        drain(0)
        repack_and_write(g0, 0)

        @pl.when(g0 + 2 < NGROUP)
        def _():
            fire(g0 + 2, 0)

        drain(1)
        repack_and_write(g0 + 1, 1)

        @pl.when(g0 + 3 < NGROUP)
        def _():
            fire(g0 + 3, 1)

        return carry

    lax.fori_loop(0, NGROUP // 2, pipe_body, 0)


def kernel(node_record, t_record, emb_table, W1, b1, W2, b2):
    node_f = node_record.astype(jnp.int32).reshape(N // 128, 128)
    t_f = t_record.reshape(N // 128, 128)
    keys = _keys_call(node_f, t_f).reshape(N)

    w1blk = jnp.zeros((ROW, 2 * ENC_DIM), jnp.float32)
    w1blk = w1blk.at[0:3, 0:ENC_DIM].set(W1).at[3:6, ENC_DIM:].set(W1)
    eye = jnp.eye(PACK, dtype=jnp.float32)
    w1bigT = jnp.kron(eye, w1blk.T)
    b1big = jnp.tile(jnp.concatenate([b1, b1]), PACK)[:, None]
    w2stack = jnp.concatenate([W2, W2], axis=0)
    w2bigT = jnp.kron(eye, w2stack.T)
    b2big = jnp.tile(2.0 * b2, PACK)[:, None]

    rows48 = [
        lax.slice(emb_table, (p, l, j), (VOCAB, l + 1, j + 1), (PACK, 1, 1))
        .reshape(1, NPACKT)
        for p in range(PACK)
        for l in range(2)
        for j in range(3)
    ]
    e48 = jnp.concatenate(rows48, axis=0)

    ftab = _tab_call(e48, w1bigT, b1big, w2bigT, b2big)

    out_b = _sc_gather(keys, ftab)
    out_t = _final_tr_call(out_b)
    out = out_t.reshape(S, ENC_DIM, B)
    return jnp.transpose(out, (2, 0, 1))


# speedup vs baseline: 3.4374x; 3.4374x over previous
import functools

import jax
import jax.numpy as jnp
from jax import lax
from jax.experimental import pallas as pl
from jax.experimental.pallas import tpu as pltpu
from jax.experimental.pallas import tpu_sc as plsc

B, S = 4096, 200
VOCAB = 1000000
ENC_DIM = 16
N = B * S
ROW = 6

PACK = 8
PK = PACK * ROW
PH = PACK * 2 * ENC_DIM
PO = PACK * ENC_DIM
BVT = 1024
GRIDT = -(-VOCAB // (PACK * BVT))
NPACKT = GRIDT * BVT
VPAD = GRIDT * PACK * BVT

NC, NS = 2, 16
NW = NC * NS
PER_W = N // NW
GKEYS = S
NGROUP = PER_W // GKEYS
HSPLIT = (104, 96)
OUTW = S * ENC_DIM

TBR = 512
GRIDF = B // TBR


def _keys_body(node_ref, t_ref, out_ref):
    s = node_ref[...] + t_ref[...].astype(jnp.int32)
    out_ref[...] = jnp.where(s >= VOCAB, s - VOCAB, s)


_keys_call = pl.pallas_call(
    _keys_body,
    out_shape=jax.ShapeDtypeStruct((N // 128, 128), jnp.int32),
)


def _mlp_body(*refs):
    xs = refs[:PACK]
    w1_ref, b1_ref, w2_ref, b2_ref, o_ref = refs[PACK:]
    x = jnp.concatenate([r[...] for r in xs], axis=0)
    h = jnp.maximum(
        jnp.dot(w1_ref[...], x, preferred_element_type=jnp.float32) + b1_ref[...],
        0.0,
    )
    o = jnp.dot(w2_ref[...], h, preferred_element_type=jnp.float32) + b2_ref[...]
    o_ref[...] = jnp.transpose(o)


def _mk_in_spec(p):
    return pl.BlockSpec((ROW, BVT), lambda i, p=p: (0, PACK * i + p))


_tab_call = pl.pallas_call(
    _mlp_body,
    grid=(GRIDT,),
    in_specs=[_mk_in_spec(p) for p in range(PACK)]
    + [
        pl.BlockSpec((PH, PK), lambda i: (0, 0)),
        pl.BlockSpec((PH, 1), lambda i: (0, 0)),
        pl.BlockSpec((PO, PH), lambda i: (0, 0)),
        pl.BlockSpec((PO, 1), lambda i: (0, 0)),
    ],
    out_specs=pl.BlockSpec((BVT, PO), lambda i: (i, 0)),
    out_shape=jax.ShapeDtypeStruct((NPACKT, PO), jnp.float32),
)


def _tr_body(x_ref, o_ref):
    o_ref[...] = jnp.transpose(x_ref[...])


_final_tr_call = pl.pallas_call(
    _tr_body,
    grid=(GRIDF,),
    in_specs=[pl.BlockSpec((TBR, OUTW), lambda i: (i, 0))],
    out_specs=pl.BlockSpec((OUTW, TBR), lambda i: (0, i)),
    out_shape=jax.ShapeDtypeStruct((OUTW, B), jnp.float32),
)


_sc_mesh = plsc.VectorSubcoreMesh(core_axis_name="c", subcore_axis_name="s")


@functools.partial(
    pl.kernel,
    out_type=jax.ShapeDtypeStruct((B, OUTW), jnp.float32),
    mesh=_sc_mesh,
    scratch_types=[
        pltpu.VMEM((PER_W,), jnp.int32),
        pltpu.VMEM((2, GKEYS), jnp.int32),
        pltpu.VMEM((2, GKEYS, PO), jnp.float32),
        pltpu.VMEM((1, OUTW), jnp.float32),
        pltpu.SemaphoreType.DMA,
        pltpu.SemaphoreType.DMA,
    ],
    compiler_params=pltpu.CompilerParams(
        use_tc_tiling_on_sc=False, needs_layout_passes=False
    ),
)
def _sc_gather(keys_hbm, ftab_hbm, out_hbm, keys_v, q_v, rows_v, stage_v, sem0, sem1):
    wid = lax.axis_index("s") * NC + lax.axis_index("c")
    key_base = wid * PER_W
    row_base = wid * NGROUP
    sems = (sem0, sem1)

    pltpu.sync_copy(keys_hbm.at[pl.ds(key_base, PER_W)], keys_v)
    iota = lax.iota(jnp.int32, 16)
    iota16 = iota * 16

    offs = [16 * t for t in range(12)] + [GKEYS - 16]

    def fire(g, buf):
        for o in offs:
            kv = keys_v[pl.ds(g * GKEYS + o, 16)]
            q_v[buf, pl.ds(o, 16)] = jnp.bitwise_or(
                lax.shift_left(lax.shift_right_logical(kv, 13), 10),
                jnp.bitwise_and(kv, 1023),
            )
        o = 0
        for sz in HSPLIT:
            pltpu.async_copy(
                ftab_hbm.at[q_v.at[buf, pl.ds(o, sz)]],
                rows_v.at[buf, pl.ds(o, sz)],
                sems[buf],
            )
            o += sz

    def drain(buf):
        o = 0
        for sz in HSPLIT:
            pltpu.make_async_copy(
                ftab_hbm.at[q_v.at[buf, pl.ds(o, sz)]],
                rows_v.at[buf, pl.ds(o, sz)],
                sems[buf],
            ).wait()
            o += sz

    def repack_and_write(g, buf):
        rows = rows_v.at[buf]
        stage = stage_v.at[0]
        for o in offs:
            kv = keys_v[pl.ds(g * GKEYS + o, 16)]
            lane_base = lax.shift_left(
                jnp.bitwise_and(lax.shift_right_logical(kv, 10), 7), 4
            )
            row_ids = iota + o
            sidx = iota16 + (o * ENC_DIM)
            for j in range(ENC_DIM):
                vals = plsc.load_gather(rows, [row_ids, lane_base + j])
                plsc.store_scatter(stage, [sidx + j], vals)
        pltpu.sync_copy(stage_v, out_hbm.at[pl.ds(row_base + g, 1)])

    fire(0, 0)
    fire(1, 1)

    def pipe_body(g2, carry):
        g0 = 2 * g2
        drain(0)
        repack_and_write(g0, 0)

        @pl.when(g0 + 2 < NGROUP)
        def _():
            fire(g0 + 2, 0)

        drain(1)
        repack_and_write(g0 + 1, 1)

        @pl.when(g0 + 3 < NGROUP)
        def _():
            fire(g0 + 3, 1)

        return carry

    lax.fori_loop(0, NGROUP // 2, pipe_body, 0)


def kernel(node_record, t_record, emb_table, W1, b1, W2, b2):
    node_f = node_record.astype(jnp.int32).reshape(N // 128, 128)
    t_f = t_record.reshape(N // 128, 128)
    keys = _keys_call(node_f, t_f).reshape(N)

    w1blk = jnp.zeros((ROW, 2 * ENC_DIM), jnp.float32)
    w1blk = w1blk.at[0:3, 0:ENC_DIM].set(W1).at[3:6, ENC_DIM:].set(W1)
    eye = jnp.eye(PACK, dtype=jnp.float32)
    w1bigT = jnp.kron(eye, w1blk.T)
    b1big = jnp.tile(jnp.concatenate([b1, b1]), PACK)[:, None]
    w2stack = jnp.concatenate([W2, W2], axis=0)
    w2bigT = jnp.kron(eye, w2stack.T)
    b2big = jnp.tile(2.0 * b2, PACK)[:, None]

    e6 = jnp.transpose(emb_table, (1, 2, 0)).reshape(2 * 3, VOCAB)
    e6 = jnp.concatenate([e6, jnp.zeros((2 * 3, VPAD - VOCAB), jnp.float32)], axis=1)

    ftab = _tab_call(*([e6] * PACK), w1bigT, b1big, w2bigT, b2big)

    out_b = _sc_gather(keys, ftab)
    out_t = _final_tr_call(out_b)
    out = out_t.reshape(S, ENC_DIM, B)
    return jnp.transpose(out, (2, 0, 1))


# speedup vs baseline: 3.4567x; 1.0056x over previous
import functools

import jax
import jax.numpy as jnp
from jax import lax
from jax.experimental import pallas as pl
from jax.experimental.pallas import tpu as pltpu
from jax.experimental.pallas import tpu_sc as plsc

B, S = 4096, 200
VOCAB = 1000000
ENC_DIM = 16
N = B * S
ROW = 6

PACK = 8
PK = PACK * ROW
PH = PACK * 2 * ENC_DIM
PO = PACK * ENC_DIM
BVT = 1024
GRIDT = -(-VOCAB // (PACK * BVT))
NPACKT = GRIDT * BVT
VPAD = GRIDT * PACK * BVT

NC, NS = 2, 16
NW = NC * NS
PER_W = N // NW
GKEYS = S
NGROUP = PER_W // GKEYS
HSPLIT = (104, 96)
OUTW = S * ENC_DIM

TBR = 512
GRIDF = B // TBR


def _keys_body(node_ref, t_ref, out_ref):
    s = node_ref[...] + t_ref[...].astype(jnp.int32)
    out_ref[...] = jnp.where(s >= VOCAB, s - VOCAB, s)


_keys_call = pl.pallas_call(
    _keys_body,
    out_shape=jax.ShapeDtypeStruct((N // 128, 128), jnp.int32),
)


def _mlp_body(*refs):
    xs = refs[:PACK]
    w1_ref, b1_ref, w2_ref, b2_ref, o_ref = refs[PACK:]
    x = jnp.concatenate([r[...] for r in xs], axis=0)
    h = jnp.maximum(
        jnp.dot(w1_ref[...], x, preferred_element_type=jnp.float32) + b1_ref[...],
        0.0,
    )
    o = jnp.dot(w2_ref[...], h, preferred_element_type=jnp.float32) + b2_ref[...]
    o_ref[...] = jnp.transpose(o)


def _mk_in_spec(p):
    return pl.BlockSpec((ROW, BVT), lambda i, p=p: (0, PACK * i + p))


_tab_call = pl.pallas_call(
    _mlp_body,
    grid=(GRIDT,),
    in_specs=[_mk_in_spec(p) for p in range(PACK)]
    + [
        pl.BlockSpec((PH, PK), lambda i: (0, 0)),
        pl.BlockSpec((PH, 1), lambda i: (0, 0)),
        pl.BlockSpec((PO, PH), lambda i: (0, 0)),
        pl.BlockSpec((PO, 1), lambda i: (0, 0)),
    ],
    out_specs=pl.BlockSpec((BVT, PO), lambda i: (i, 0)),
    out_shape=jax.ShapeDtypeStruct((NPACKT, PO), jnp.float32),
)


def _tr_body(x_ref, o_ref):
    o_ref[...] = jnp.transpose(x_ref[...])


_final_tr_call = pl.pallas_call(
    _tr_body,
    grid=(GRIDF,),
    in_specs=[pl.BlockSpec((TBR, OUTW), lambda i: (i, 0))],
    out_specs=pl.BlockSpec((OUTW, TBR), lambda i: (0, i)),
    out_shape=jax.ShapeDtypeStruct((OUTW, B), jnp.float32),
)


_sc_mesh = plsc.VectorSubcoreMesh(core_axis_name="c", subcore_axis_name="s")


@functools.partial(
    pl.kernel,
    out_type=jax.ShapeDtypeStruct((B, OUTW), jnp.float32),
    mesh=_sc_mesh,
    scratch_types=[
        pltpu.VMEM((PER_W,), jnp.int32),
        pltpu.VMEM((2, GKEYS), jnp.int32),
        pltpu.VMEM((2, GKEYS, PO), jnp.float32),
        pltpu.VMEM((2, 1, OUTW), jnp.float32),
        pltpu.SemaphoreType.DMA,
        pltpu.SemaphoreType.DMA,
        pltpu.SemaphoreType.DMA,
        pltpu.SemaphoreType.DMA,
    ],
    compiler_params=pltpu.CompilerParams(
        use_tc_tiling_on_sc=False, needs_layout_passes=False
    ),
)
def _sc_gather(
    keys_hbm, ftab_hbm, out_hbm, keys_v, q_v, rows_v, stage_v,
    sem0, sem1, osem0, osem1,
):
    wid = lax.axis_index("s") * NC + lax.axis_index("c")
    key_base = wid * PER_W
    row_base = wid * NGROUP
    sems = (sem0, sem1)

    pltpu.sync_copy(keys_hbm.at[pl.ds(key_base, PER_W)], keys_v)
    iota = lax.iota(jnp.int32, 16)
    iota16 = iota * 16

    offs = [16 * t for t in range(12)] + [GKEYS - 16]

    def fire(g, buf):
        for o in offs:
            kv = keys_v[pl.ds(g * GKEYS + o, 16)]
            q_v[buf, pl.ds(o, 16)] = jnp.bitwise_or(
                lax.shift_left(lax.shift_right_logical(kv, 13), 10),
                jnp.bitwise_and(kv, 1023),
            )
        o = 0
        for sz in HSPLIT:
            pltpu.async_copy(
                ftab_hbm.at[q_v.at[buf, pl.ds(o, sz)]],
                rows_v.at[buf, pl.ds(o, sz)],
                sems[buf],
            )
            o += sz

    def drain(buf):
        o = 0
        for sz in HSPLIT:
            pltpu.make_async_copy(
                ftab_hbm.at[q_v.at[buf, pl.ds(o, sz)]],
                rows_v.at[buf, pl.ds(o, sz)],
                sems[buf],
            ).wait()
            o += sz

    osems = (osem0, osem1)

    def repack_and_write(g, buf, first):
        rows = rows_v.at[buf]
        stage = stage_v.at[buf, 0]
        if not first:
            pltpu.make_async_copy(
                stage_v.at[buf], out_hbm.at[pl.ds(row_base + g, 1)], osems[buf]
            ).wait()
        for o in offs:
            kv = keys_v[pl.ds(g * GKEYS + o, 16)]
            lane_base = lax.shift_left(
                jnp.bitwise_and(lax.shift_right_logical(kv, 10), 7), 4
            )
            row_ids = iota + o
            sidx = iota16 + (o * ENC_DIM)
            for j in range(ENC_DIM):
                vals = plsc.load_gather(rows, [row_ids, lane_base + j])
                plsc.store_scatter(stage, [sidx + j], vals)
        pltpu.async_copy(
            stage_v.at[buf], out_hbm.at[pl.ds(row_base + g, 1)], osems[buf]
        )

    fire(0, 0)
    fire(1, 1)
    drain(0)
    repack_and_write(0, 0, True)
    fire(2, 0)
    drain(1)
    repack_and_write(1, 1, True)
    fire(3, 1)

    def pipe_body(g2, carry):
        g0 = 2 * g2
        drain(0)
        repack_and_write(g0, 0, False)

        @pl.when(g0 + 2 < NGROUP)
        def _():
            fire(g0 + 2, 0)

        drain(1)
        repack_and_write(g0 + 1, 1, False)

        @pl.when(g0 + 3 < NGROUP)
        def _():
            fire(g0 + 3, 1)

        return carry

    lax.fori_loop(1, NGROUP // 2, pipe_body, 0)

    for buf in range(2):
        pltpu.make_async_copy(
            stage_v.at[buf], out_hbm.at[pl.ds(row_base, 1)], osems[buf]
        ).wait()


def kernel(node_record, t_record, emb_table, W1, b1, W2, b2):
    node_f = node_record.astype(jnp.int32).reshape(N // 128, 128)
    t_f = t_record.reshape(N // 128, 128)
    keys = _keys_call(node_f, t_f).reshape(N)

    w1blk = jnp.zeros((ROW, 2 * ENC_DIM), jnp.float32)
    w1blk = w1blk.at[0:3, 0:ENC_DIM].set(W1).at[3:6, ENC_DIM:].set(W1)
    eye = jnp.eye(PACK, dtype=jnp.float32)
    w1bigT = jnp.kron(eye, w1blk.T)
    b1big = jnp.tile(jnp.concatenate([b1, b1]), PACK)[:, None]
    w2stack = jnp.concatenate([W2, W2], axis=0)
    w2bigT = jnp.kron(eye, w2stack.T)
    b2big = jnp.tile(2.0 * b2, PACK)[:, None]

    e6 = jnp.transpose(emb_table, (1, 2, 0)).reshape(2 * 3, VOCAB)
    e6 = lax.dynamic_update_slice(jnp.zeros((2 * 3, VPAD), jnp.float32), e6, (0, 0))

    ftab = _tab_call(*([e6] * PACK), w1bigT, b1big, w2bigT, b2big)

    out_b = _sc_gather(keys, ftab)
    out_t = _final_tr_call(out_b)
    out = out_t.reshape(S, ENC_DIM, B)
    return jnp.transpose(out, (2, 0, 1))
